# Initial kernel scaffold; baseline (speedup 1.0000x reference)
import jax
import jax.numpy as jnp
from jax.experimental import pallas as pl


def kernel(atom_type, aa_type, aa_pos, mask, atom_table, residue_table, pos_table, W1, b1, W2, b2):
    raise NotImplementedError("write your pallas kernel here")


# speedup vs baseline: 11.5448x; 11.5448x over previous
import functools

import jax
import jax.numpy as jnp
from jax.experimental import pallas as pl
from jax.experimental.pallas import tpu as pltpu

C = 512
PAD = 64
V = 3 * PAD
M = 2048


def _prefuse_body(tabs_ref, w1_ref, g_ref):
    for k in range(3):
        t = tabs_ref[k * PAD:(k + 1) * PAD, :]
        w = w1_ref[k * C:(k + 1) * C, :]
        g = jnp.dot(t, w, preferred_element_type=jnp.float32)
        g_ref[k * PAD:(k + 1) * PAD, :] = g.astype(jnp.bfloat16)


def _mlp_body(ia_ref, ir_ref, ip_ref, mask_ref, g_ref, b1_ref, w2_ref,
              b2_ref, out_ref):
    ia = ia_ref[0, 0, :]
    ir = ir_ref[0, 0, :]
    ip = ip_ref[0, 0, :]
    cols = jax.lax.broadcasted_iota(jnp.int32, (M, V), 1)
    mh = ((cols == ia[:, None]).astype(jnp.bfloat16)
          + (cols == ir[:, None]).astype(jnp.bfloat16)
          + (cols == ip[:, None]).astype(jnp.bfloat16))
    y = jnp.dot(mh, g_ref[...], preferred_element_type=jnp.float32)
    y = y + b1_ref[...]
    h = (y * 0.5 * (1.0 + jax.lax.erf(y * 0.7071067811865476))).astype(jnp.bfloat16)
    out = jnp.dot(h, w2_ref[...], preferred_element_type=jnp.float32)
    out = out + b2_ref[...]
    out_ref[...] = out * mask_ref[0, 0, :][:, None]


def kernel(atom_type, aa_type, aa_pos, mask, atom_table, residue_table,
           pos_table, W1, b1, W2, b2):
    B, N = atom_type.shape
    T = B * N
    nb = T // M

    tabs = jnp.zeros((V, C), jnp.float32)
    tabs = tabs.at[0:55].set(atom_table)
    tabs = tabs.at[PAD:PAD + 21].set(residue_table)
    tabs = tabs.at[2 * PAD:2 * PAD + 24].set(pos_table)

    g = pl.pallas_call(
        _prefuse_body,
        out_shape=jax.ShapeDtypeStruct((V, C), jnp.bfloat16),
    )(tabs, W1)

    ia = atom_type.reshape(nb, 1, M).astype(jnp.int32)
    ir = (aa_type.reshape(nb, 1, M) + PAD).astype(jnp.int32)
    ip = (aa_pos.reshape(nb, 1, M) + 2 * PAD).astype(jnp.int32)
    mask_f = mask.reshape(nb, 1, M).astype(jnp.float32)

    idx_spec = pl.BlockSpec((1, 1, M), lambda i: (i, 0, 0))
    full = lambda shape: pl.BlockSpec(shape, lambda i: (0,) * len(shape))

    out = pl.pallas_call(
        _mlp_body,
        grid=(nb,),
        in_specs=[idx_spec, idx_spec, idx_spec, idx_spec,
                  full((V, C)), full((1, C)), full((C, C)), full((1, C))],
        out_specs=pl.BlockSpec((M, C), lambda i: (i, 0)),
        out_shape=jax.ShapeDtypeStruct((T, C), jnp.float32),
    )(ia, ir, ip, mask_f, g, b1.reshape(1, C), W2.astype(jnp.bfloat16),
      b2.reshape(1, C))

    return out.reshape(B, N, C)
